# Initial kernel scaffold; baseline (speedup 1.0000x reference)
import jax
import jax.numpy as jnp
from jax.experimental import pallas as pl


def kernel(x, edge_index, W, attn_weights):
    raise NotImplementedError("write your pallas kernel here")


# speedup vs baseline: 6.8734x; 6.8734x over previous
import functools

import jax
import jax.numpy as jnp
from jax import lax
from jax.experimental import pallas as pl
from jax.experimental.pallas import tpu as pltpu
from jax.experimental.pallas import tpu_sc as plsc

NC = 2
NS = 16
LANES = 16
HEADS = 4


def _prep_body(x_ref, w_ref, a_ref, h_ref, s_ref, t_ref):
    h = jnp.dot(x_ref[...], w_ref[...], preferred_element_type=jnp.float32)
    h_ref[...] = h
    a = a_ref[...]
    s_ref[...] = jnp.sum(h * a[0][None, :], axis=1)[None, None, :]
    t_ref[...] = jnp.sum(h * a[1][None, :], axis=1)[None, None, :]


def _tc_prep(x, W, a_pair, block_n):
    n, d_in = x.shape
    d_out = W.shape[1]
    grid = n // block_n
    return pl.pallas_call(
        _prep_body,
        grid=(grid,),
        in_specs=[
            pl.BlockSpec((block_n, d_in), lambda i: (i, 0)),
            pl.BlockSpec((d_in, d_out), lambda i: (0, 0)),
            pl.BlockSpec((2, d_out), lambda i: (0, 0)),
        ],
        out_specs=[
            pl.BlockSpec((block_n, d_out), lambda i: (i, 0)),
            pl.BlockSpec((1, 1, block_n), lambda i: (i, 0, 0)),
            pl.BlockSpec((1, 1, block_n), lambda i: (i, 0, 0)),
        ],
        out_shape=[
            jax.ShapeDtypeStruct((n, d_out), jnp.float32),
            jax.ShapeDtypeStruct((grid, 1, block_n), jnp.float32),
            jax.ShapeDtypeStruct((grid, 1, block_n), jnp.float32),
        ],
    )(x, W, a_pair)


def _sc_gat(h, s, t, ei4, ej4, n, n_pad, d, chunks_per_tile, k):
    rows_per_tile = n_pad // NS
    zc = rows_per_tile // k
    assert rows_per_tile % k == 0

    mesh = plsc.VectorSubcoreMesh(core_axis_name="c", subcore_axis_name="s")

    @functools.partial(
        pl.kernel,
        mesh=mesh,
        out_type=[
            jax.ShapeDtypeStruct((NC, n_pad, d), jnp.float32),
            jax.ShapeDtypeStruct((NC * NS, 1, LANES), jnp.float32),
        ],
        scratch_types=[
            pltpu.VMEM((n,), jnp.float32),
            pltpu.VMEM((n,), jnp.float32),
            pltpu.VMEM((1, k), jnp.int32),
            pltpu.VMEM((1, k), jnp.int32),
            pltpu.VMEM((1, k), jnp.float32),
            pltpu.VMEM((k, d), jnp.float32),
            pltpu.VMEM((1, LANES), jnp.float32),
            pltpu.VMEM_SHARED((n_pad, d), jnp.float32),
            pltpu.SemaphoreType.DMA,
        ],
        compiler_params=pltpu.CompilerParams(needs_layout_passes=False),
    )
    def sc_kernel(h_hbm, s_hbm, t_hbm, ei_hbm, ej_hbm, out_hbm, z_hbm,
                  s_v, t_v, ei_c, ej_c, w_c, rows_v, zbuf_v, acc, sem):
        cid = lax.axis_index("c")
        sid = lax.axis_index("s")
        wid = cid * NS + sid

        pltpu.sync_copy(s_hbm, s_v)
        pltpu.sync_copy(t_hbm, t_v)

        def zbody(r, _):
            for u in range(d // LANES):
                rows_v[r, pl.ds(u * LANES, LANES)] = jnp.zeros((LANES,), jnp.float32)
            return 0
        lax.fori_loop(0, k, zbody, 0)
        row0 = sid * rows_per_tile
        for kk in range(zc):
            pltpu.sync_copy(rows_v, acc.at[pl.ds(row0 + kk * k, k)])

        plsc.subcore_barrier()

        base = wid * chunks_per_tile

        def cbody(j, z):
            pltpu.sync_copy(ei_hbm.at[base + j], ei_c)
            pltpu.sync_copy(ej_hbm.at[base + j], ej_c)

            for u in range(k // LANES):
                sl = pl.ds(u * LANES, LANES)
                ii = ei_c[0, sl]
                jj = ej_c[0, sl]
                l = plsc.load_gather(s_v, [ii]) + plsc.load_gather(t_v, [jj])
                l = jnp.where(l > 0, l, l * jnp.float32(0.2))
                w = jnp.exp(l)
                w_c[0, sl] = w
                z = z + w

            pltpu.async_copy(h_hbm.at[ej_c.at[0]], rows_v, sem).wait()

            def rbody(g, _):
                wvec = w_c[0, pl.ds(g * LANES, LANES)]
                for rl in range(LANES):
                    ws = wvec[rl]
                    r = g * LANES + rl
                    for u in range(d // LANES):
                        sl2 = pl.ds(u * LANES, LANES)
                        rows_v[r, sl2] = rows_v[r, sl2] * ws
                return 0
            lax.fori_loop(0, k // LANES, rbody, 0)

            pltpu.sync_copy(rows_v, acc.at[ei_c.at[0]], add=True)
            return z

        z = lax.fori_loop(0, chunks_per_tile, cbody,
                          jnp.zeros((LANES,), jnp.float32))
        zbuf_v[0, pl.ds(0, LANES)] = z
        pltpu.sync_copy(zbuf_v, z_hbm.at[wid])

        plsc.subcore_barrier()
        pltpu.sync_copy(acc.at[pl.ds(row0, rows_per_tile)],
                        out_hbm.at[cid, pl.ds(row0, rows_per_tile)])

    return sc_kernel(h, s, t, ei4, ej4)


def _combine_body(acc_ref, z_ref, o_ref):
    p = acc_ref[0] + acc_ref[1]
    inv = jnp.float32(1.0) / jnp.sum(z_ref[...])
    v = p * inv
    o_ref[...] = jnp.concatenate([v] * HEADS, axis=1)


def _tc_combine(acc, z, n, d, block_n):
    grid = n // block_n
    return pl.pallas_call(
        _combine_body,
        grid=(grid,),
        in_specs=[
            pl.BlockSpec((NC, block_n, d), lambda i: (0, i, 0)),
            pl.BlockSpec((NC * NS, 1, LANES), lambda i: (0, 0, 0)),
        ],
        out_specs=pl.BlockSpec((block_n, HEADS * d), lambda i: (i, 0)),
        out_shape=jax.ShapeDtypeStruct((n, HEADS * d), jnp.float32),
    )(acc, z)


def kernel(x, edge_index, W, attn_weights):
    n, d_in = x.shape
    d = W.shape[1]
    e = edge_index.shape[1]

    k = 80
    chunks_per_tile = e // (NC * NS * k)
    assert chunks_per_tile * NC * NS * k == e

    a_pair = attn_weights.reshape(2, d)
    ei4 = edge_index[0].reshape(e // k, 1, k)
    ej4 = edge_index[1].reshape(e // k, 1, k)

    rpt = -(-n // NS)
    rpt = -(-rpt // 128) * 128
    n_pad = rpt * NS

    block_n = 1000
    h, s3, t3 = _tc_prep(x, W, a_pair, block_n)
    s = s3.reshape(n)
    t = t3.reshape(n)

    acc, z = _sc_gat(h, s, t, ei4, ej4, n, n_pad, d, chunks_per_tile, k)
    return _tc_combine(acc, z, n, d, block_n)


# speedup vs baseline: 11.3348x; 1.6491x over previous
import functools

import jax
import jax.numpy as jnp
from jax import lax
from jax.experimental import pallas as pl
from jax.experimental.pallas import tpu as pltpu
from jax.experimental.pallas import tpu_sc as plsc

NC = 2
NS = 16
LANES = 16
HEADS = 4


def _prep_body(x_ref, w_ref, a_ref, h_ref, s_ref, t_ref):
    h = jnp.dot(x_ref[...], w_ref[...], preferred_element_type=jnp.float32)
    h_ref[...] = h
    a = a_ref[...]
    s_ref[...] = jnp.sum(h * a[0][None, :], axis=1)[None, None, :]
    t_ref[...] = jnp.sum(h * a[1][None, :], axis=1)[None, None, :]


def _tc_prep(x, W, a_pair, block_n):
    n, d_in = x.shape
    d_out = W.shape[1]
    grid = n // block_n
    return pl.pallas_call(
        _prep_body,
        grid=(grid,),
        in_specs=[
            pl.BlockSpec((block_n, d_in), lambda i: (i, 0)),
            pl.BlockSpec((d_in, d_out), lambda i: (0, 0)),
            pl.BlockSpec((2, d_out), lambda i: (0, 0)),
        ],
        out_specs=[
            pl.BlockSpec((block_n, d_out), lambda i: (i, 0)),
            pl.BlockSpec((1, 1, block_n), lambda i: (i, 0, 0)),
            pl.BlockSpec((1, 1, block_n), lambda i: (i, 0, 0)),
        ],
        out_shape=[
            jax.ShapeDtypeStruct((n, d_out), jnp.float32),
            jax.ShapeDtypeStruct((grid, 1, block_n), jnp.float32),
            jax.ShapeDtypeStruct((grid, 1, block_n), jnp.float32),
        ],
    )(x, W, a_pair)


def _sc_gat(h, s, t, ei4, ej4, n, n_pad, d, chunks_per_tile, k):
    rows_per_tile = n_pad // NS
    zc = rows_per_tile // k
    assert rows_per_tile % k == 0

    mesh = plsc.VectorSubcoreMesh(core_axis_name="c", subcore_axis_name="s")

    @functools.partial(
        pl.kernel,
        mesh=mesh,
        out_type=[
            jax.ShapeDtypeStruct((NC, n_pad, d), jnp.float32),
            jax.ShapeDtypeStruct((NC * NS, 1, LANES), jnp.float32),
        ],
        scratch_types=[
            pltpu.VMEM((n,), jnp.float32),
            pltpu.VMEM((n,), jnp.float32),
            pltpu.VMEM((2, k), jnp.int32),
            pltpu.VMEM((2, k), jnp.int32),
            pltpu.VMEM((2, k), jnp.int32),
            pltpu.VMEM((2, k), jnp.float32),
            pltpu.VMEM((2, k, d), jnp.float32),
            pltpu.VMEM((1, LANES), jnp.float32),
            pltpu.VMEM_SHARED((n_pad, d), jnp.float32),
            pltpu.SemaphoreType.DMA,
            pltpu.SemaphoreType.DMA,
            pltpu.SemaphoreType.DMA,
            pltpu.SemaphoreType.DMA,
            pltpu.SemaphoreType.DMA,
            pltpu.SemaphoreType.DMA,
            pltpu.SemaphoreType.DMA,
            pltpu.SemaphoreType.DMA,
        ],
        compiler_params=pltpu.CompilerParams(needs_layout_passes=False),
    )
    def sc_kernel(h_hbm, s_hbm, t_hbm, ei_hbm, ej_hbm, out_hbm, z_hbm,
                  s_v, t_v, ei_c, ej_c, sci, w_c, rows_v, zbuf_v, acc,
                  si0, si1, sj0, sj1, sr0, sr1, ss0, ss1):
        si = (si0, si1)
        sj = (sj0, sj1)
        sr = (sr0, sr1)
        ss = (ss0, ss1)
        cid = lax.axis_index("c")
        sid = lax.axis_index("s")
        wid = cid * NS + sid
        base = wid * chunks_per_tile

        for b in range(2):
            pltpu.async_copy(ei_hbm.at[base + b], ei_c.at[pl.ds(b, 1)], si[b])
            pltpu.async_copy(ej_hbm.at[base + b], ej_c.at[pl.ds(b, 1)], sj[b])
        pltpu.sync_copy(s_hbm, s_v)
        pltpu.sync_copy(t_hbm, t_v)

        def zbody(r, _):
            for u in range(d // LANES):
                rows_v[0, r, pl.ds(u * LANES, LANES)] = jnp.zeros(
                    (LANES,), jnp.float32)
            return 0
        lax.fori_loop(0, k, zbody, 0)
        row0 = sid * rows_per_tile
        for kk in range(zc):
            pltpu.sync_copy(rows_v.at[0], acc.at[pl.ds(row0 + kk * k, k)])

        plsc.subcore_barrier()

        def chunk_step(g, j, b, z, prefetch):
            pltpu.make_async_copy(
                ei_hbm.at[base], ei_c.at[pl.ds(b, 1)], si[b]).wait()
            pltpu.make_async_copy(
                ej_hbm.at[base], ej_c.at[pl.ds(b, 1)], sj[b]).wait()

            @pl.when(g >= 1)
            def _():
                pltpu.make_async_copy(
                    rows_v.at[b], acc.at[sci.at[b]], ss[b]).wait()

            gather = pltpu.async_copy(
                h_hbm.at[ej_c.at[b]], rows_v.at[b], sr[b])

            for u in range(k // LANES):
                sl = pl.ds(u * LANES, LANES)
                ii = ei_c[b, sl]
                jj = ej_c[b, sl]
                l = plsc.load_gather(s_v, [ii]) + plsc.load_gather(t_v, [jj])
                l = jnp.where(l > 0, l, l * jnp.float32(0.2))
                w = jnp.exp(l)
                w_c[b, sl] = w
                z = z + w
                sci[b, sl] = ii

            gather.wait()
            if prefetch:
                @pl.when(j + 2 < chunks_per_tile)
                def _():
                    pltpu.async_copy(
                        ei_hbm.at[base + j + 2], ei_c.at[pl.ds(b, 1)], si[b])
                    pltpu.async_copy(
                        ej_hbm.at[base + j + 2], ej_c.at[pl.ds(b, 1)], sj[b])

            def rbody(gg, _):
                wvec = w_c[b, pl.ds(gg * LANES, LANES)]
                for rl in range(LANES):
                    ws = wvec[rl]
                    r = gg * LANES + rl
                    for u in range(d // LANES):
                        sl2 = pl.ds(u * LANES, LANES)
                        rows_v[b, r, sl2] = rows_v[b, r, sl2] * ws
                return 0
            lax.fori_loop(0, k // LANES, rbody, 0)

            pltpu.async_copy(rows_v.at[b], acc.at[sci.at[b]], ss[b], add=True)
            return z

        npairs = chunks_per_tile // 2
        def pair_body(g, z):
            for b in range(2):
                z = chunk_step(g, g * 2 + b, b, z, True)
            return z
        z = lax.fori_loop(0, npairs, pair_body,
                          jnp.zeros((LANES,), jnp.float32))
        if chunks_per_tile % 2:
            z = chunk_step(npairs, chunks_per_tile - 1, 0, z, False)
            pltpu.make_async_copy(
                rows_v.at[0], acc.at[sci.at[0]], ss[0]).wait()
            pltpu.make_async_copy(
                rows_v.at[1], acc.at[sci.at[1]], ss[1]).wait()
        else:
            for b in range(2):
                pltpu.make_async_copy(
                    rows_v.at[b], acc.at[sci.at[b]], ss[b]).wait()

        zbuf_v[0, pl.ds(0, LANES)] = z
        pltpu.sync_copy(zbuf_v, z_hbm.at[wid])

        plsc.subcore_barrier()
        pltpu.sync_copy(acc.at[pl.ds(row0, rows_per_tile)],
                        out_hbm.at[cid, pl.ds(row0, rows_per_tile)])

    return sc_kernel(h, s, t, ei4, ej4)


def _combine_body(acc_ref, z_ref, o_ref):
    p = acc_ref[0] + acc_ref[1]
    inv = jnp.float32(1.0) / jnp.sum(z_ref[...])
    v = p * inv
    o_ref[...] = jnp.concatenate([v] * HEADS, axis=1)


def _tc_combine(acc, z, n, d, block_n):
    grid = n // block_n
    return pl.pallas_call(
        _combine_body,
        grid=(grid,),
        in_specs=[
            pl.BlockSpec((NC, block_n, d), lambda i: (0, i, 0)),
            pl.BlockSpec((NC * NS, 1, LANES), lambda i: (0, 0, 0)),
        ],
        out_specs=pl.BlockSpec((block_n, HEADS * d), lambda i: (i, 0)),
        out_shape=jax.ShapeDtypeStruct((n, HEADS * d), jnp.float32),
    )(acc, z)


def kernel(x, edge_index, W, attn_weights):
    n, d_in = x.shape
    d = W.shape[1]
    e = edge_index.shape[1]

    k = 80
    chunks_per_tile = e // (NC * NS * k)
    assert chunks_per_tile * NC * NS * k == e

    a_pair = attn_weights.reshape(2, d)
    ei4 = edge_index[0].reshape(e // k, 1, k)
    ej4 = edge_index[1].reshape(e // k, 1, k)

    rpt = -(-n // NS)
    rpt = -(-rpt // 128) * 128
    n_pad = rpt * NS

    block_n = 1000
    h, s3, t3 = _tc_prep(x, W, a_pair, block_n)
    s = s3.reshape(n)
    t = t3.reshape(n)

    acc, z = _sc_gat(h, s, t, ei4, ej4, n, n_pad, d, chunks_per_tile, k)
    return _tc_combine(acc, z, n, d, block_n)


# speedup vs baseline: 13.7079x; 1.2094x over previous
import functools

import jax
import jax.numpy as jnp
from jax import lax
from jax.experimental import pallas as pl
from jax.experimental.pallas import tpu as pltpu
from jax.experimental.pallas import tpu_sc as plsc

NC = 2
NS = 16
LANES = 16
HEADS = 4


def _prep_body(x_ref, w_ref, a_ref, h_ref, s_ref, t_ref):
    h = jnp.dot(x_ref[...], w_ref[...], preferred_element_type=jnp.float32)
    h_ref[...] = h
    a = a_ref[...]
    s_ref[...] = jnp.sum(h * a[0][None, :], axis=1)[None, None, :]
    t_ref[...] = jnp.sum(h * a[1][None, :], axis=1)[None, None, :]


def _tc_prep(x, W, a_pair, block_n):
    n, d_in = x.shape
    d_out = W.shape[1]
    grid = n // block_n
    return pl.pallas_call(
        _prep_body,
        grid=(grid,),
        in_specs=[
            pl.BlockSpec((block_n, d_in), lambda i: (i, 0)),
            pl.BlockSpec((d_in, d_out), lambda i: (0, 0)),
            pl.BlockSpec((2, d_out), lambda i: (0, 0)),
        ],
        out_specs=[
            pl.BlockSpec((block_n, d_out), lambda i: (i, 0)),
            pl.BlockSpec((1, 1, block_n), lambda i: (i, 0, 0)),
            pl.BlockSpec((1, 1, block_n), lambda i: (i, 0, 0)),
        ],
        out_shape=[
            jax.ShapeDtypeStruct((n, d_out), jnp.float32),
            jax.ShapeDtypeStruct((grid, 1, block_n), jnp.float32),
            jax.ShapeDtypeStruct((grid, 1, block_n), jnp.float32),
        ],
    )(x, W, a_pair)


def _sc_gat(h, s, t, ei4, ej4, n, n_pad, d, chunks_per_tile, k):
    rows_per_tile = n_pad // NS
    zc = rows_per_tile // k
    assert rows_per_tile % k == 0

    mesh = plsc.VectorSubcoreMesh(core_axis_name="c", subcore_axis_name="s")

    @functools.partial(
        pl.kernel,
        mesh=mesh,
        out_type=[
            jax.ShapeDtypeStruct((NC, n_pad, d), jnp.float32),
            jax.ShapeDtypeStruct((NC * NS, 1, LANES), jnp.float32),
        ],
        scratch_types=[
            pltpu.VMEM((n,), jnp.float32),
            pltpu.VMEM((n,), jnp.float32),
            pltpu.VMEM((2, k), jnp.int32),
            pltpu.VMEM((2, k), jnp.int32),
            pltpu.VMEM((2, k), jnp.int32),
            pltpu.VMEM((2, k), jnp.float32),
            pltpu.VMEM((2, k, d), jnp.float32),
            pltpu.VMEM((1, LANES), jnp.float32),
            pltpu.VMEM_SHARED((n_pad, d), jnp.float32),
            pltpu.SemaphoreType.DMA,
            pltpu.SemaphoreType.DMA,
            pltpu.SemaphoreType.DMA,
            pltpu.SemaphoreType.DMA,
            pltpu.SemaphoreType.DMA,
            pltpu.SemaphoreType.DMA,
            pltpu.SemaphoreType.DMA,
            pltpu.SemaphoreType.DMA,
        ],
        compiler_params=pltpu.CompilerParams(needs_layout_passes=False),
    )
    def sc_kernel(h_hbm, s_hbm, t_hbm, ei_hbm, ej_hbm, out_hbm, z_hbm,
                  s_v, t_v, ei_c, ej_c, sci, w_c, rows_v, zbuf_v, acc,
                  si0, si1, sj0, sj1, sr0, sr1, ss0, ss1):
        si = (si0, si1)
        sj = (sj0, sj1)
        sr = (sr0, sr1)
        ss = (ss0, ss1)
        cid = lax.axis_index("c")
        sid = lax.axis_index("s")
        wid = cid * NS + sid
        base = wid * chunks_per_tile

        for b in range(2):
            pltpu.async_copy(ei_hbm.at[base + b], ei_c.at[pl.ds(b, 1)], si[b])
            pltpu.async_copy(ej_hbm.at[base + b], ej_c.at[pl.ds(b, 1)], sj[b])
        pltpu.sync_copy(s_hbm, s_v)
        pltpu.sync_copy(t_hbm, t_v)

        def zbody(r, _):
            for u in range(d // LANES):
                rows_v[0, r, pl.ds(u * LANES, LANES)] = jnp.zeros(
                    (LANES,), jnp.float32)
            return 0
        lax.fori_loop(0, k, zbody, 0)
        row0 = sid * rows_per_tile
        for kk in range(zc):
            pltpu.sync_copy(rows_v.at[0], acc.at[pl.ds(row0 + kk * k, k)])

        plsc.subcore_barrier()

        pltpu.make_async_copy(
            ei_hbm.at[base], ei_c.at[pl.ds(0, 1)], si[0]).wait()
        pltpu.make_async_copy(
            ej_hbm.at[base], ej_c.at[pl.ds(0, 1)], sj[0]).wait()
        pltpu.async_copy(h_hbm.at[ej_c.at[0]], rows_v.at[0], sr[0])

        def chunk_step(j, b, z):
            b2 = 1 - b

            for u in range(k // LANES):
                sl = pl.ds(u * LANES, LANES)
                ii = ei_c[b, sl]
                jj = ej_c[b, sl]
                l = plsc.load_gather(s_v, [ii]) + plsc.load_gather(t_v, [jj])
                l = jnp.where(l > 0, l, l * jnp.float32(0.2))
                w = jnp.exp(l)
                w_c[b, sl] = w
                z = z + w
                sci[b, sl] = ii

            pltpu.make_async_copy(
                h_hbm.at[ej_c.at[b]], rows_v.at[b], sr[b]).wait()

            @pl.when(j + 2 < chunks_per_tile)
            def _():
                pltpu.async_copy(
                    ei_hbm.at[base + j + 2], ei_c.at[pl.ds(b, 1)], si[b])
                pltpu.async_copy(
                    ej_hbm.at[base + j + 2], ej_c.at[pl.ds(b, 1)], sj[b])

            @pl.when(j + 1 < chunks_per_tile)
            def _():
                pltpu.make_async_copy(
                    ei_hbm.at[base], ei_c.at[pl.ds(b2, 1)], si[b2]).wait()
                pltpu.make_async_copy(
                    ej_hbm.at[base], ej_c.at[pl.ds(b2, 1)], sj[b2]).wait()

                @pl.when(j >= 1)
                def _():
                    pltpu.make_async_copy(
                        rows_v.at[b2], acc.at[sci.at[b2]], ss[b2]).wait()
                pltpu.async_copy(h_hbm.at[ej_c.at[b2]], rows_v.at[b2], sr[b2])

            def rbody(gg, _):
                wvec = w_c[b, pl.ds(gg * LANES, LANES)]
                for rl in range(LANES):
                    ws = wvec[rl]
                    r = gg * LANES + rl
                    for u in range(d // LANES):
                        sl2 = pl.ds(u * LANES, LANES)
                        rows_v[b, r, sl2] = rows_v[b, r, sl2] * ws
                return 0
            lax.fori_loop(0, k // LANES, rbody, 0)

            pltpu.async_copy(rows_v.at[b], acc.at[sci.at[b]], ss[b], add=True)
            return z

        npairs = chunks_per_tile // 2
        def pair_body(g, z):
            z = chunk_step(g * 2, 0, z)
            z = chunk_step(g * 2 + 1, 1, z)
            return z
        z = lax.fori_loop(0, npairs, pair_body,
                          jnp.zeros((LANES,), jnp.float32))
        if chunks_per_tile % 2:
            z = chunk_step(chunks_per_tile - 1, 0, z)
        for b in range(2):
            pltpu.make_async_copy(
                rows_v.at[b], acc.at[sci.at[b]], ss[b]).wait()

        zbuf_v[0, pl.ds(0, LANES)] = z
        pltpu.sync_copy(zbuf_v, z_hbm.at[wid])

        plsc.subcore_barrier()
        pltpu.sync_copy(acc.at[pl.ds(row0, rows_per_tile)],
                        out_hbm.at[cid, pl.ds(row0, rows_per_tile)])

    return sc_kernel(h, s, t, ei4, ej4)


def _combine_body(acc_ref, z_ref, o_ref):
    p = acc_ref[0] + acc_ref[1]
    inv = jnp.float32(1.0) / jnp.sum(z_ref[...])
    v = p * inv
    o_ref[...] = jnp.concatenate([v] * HEADS, axis=1)


def _tc_combine(acc, z, n, d, block_n):
    grid = n // block_n
    return pl.pallas_call(
        _combine_body,
        grid=(grid,),
        in_specs=[
            pl.BlockSpec((NC, block_n, d), lambda i: (0, i, 0)),
            pl.BlockSpec((NC * NS, 1, LANES), lambda i: (0, 0, 0)),
        ],
        out_specs=pl.BlockSpec((block_n, HEADS * d), lambda i: (i, 0)),
        out_shape=jax.ShapeDtypeStruct((n, HEADS * d), jnp.float32),
    )(acc, z)


def kernel(x, edge_index, W, attn_weights):
    n, d_in = x.shape
    d = W.shape[1]
    e = edge_index.shape[1]

    k = 80
    chunks_per_tile = e // (NC * NS * k)
    assert chunks_per_tile * NC * NS * k == e

    a_pair = attn_weights.reshape(2, d)
    ei4 = edge_index[0].reshape(e // k, 1, k)
    ej4 = edge_index[1].reshape(e // k, 1, k)

    rpt = -(-n // NS)
    rpt = -(-rpt // 128) * 128
    n_pad = rpt * NS

    block_n = 1000
    h, s3, t3 = _tc_prep(x, W, a_pair, block_n)
    s = s3.reshape(n)
    t = t3.reshape(n)

    acc, z = _sc_gat(h, s, t, ei4, ej4, n, n_pad, d, chunks_per_tile, k)
    return _tc_combine(acc, z, n, d, block_n)


# speedup vs baseline: 13.7561x; 1.0035x over previous
import functools

import jax
import jax.numpy as jnp
from jax import lax
from jax.experimental import pallas as pl
from jax.experimental.pallas import tpu as pltpu
from jax.experimental.pallas import tpu_sc as plsc

NC = 2
NS = 16
LANES = 16
HEADS = 4


def _prep_body(x_ref, w_ref, a_ref, h_ref, s_ref, t_ref):
    h = jnp.dot(x_ref[...], w_ref[...], preferred_element_type=jnp.float32)
    h_ref[...] = h
    a = a_ref[...]
    s_ref[...] = jnp.sum(h * a[0][None, :], axis=1)[None, None, :]
    t_ref[...] = jnp.sum(h * a[1][None, :], axis=1)[None, None, :]


def _tc_prep(x, W, a_pair, block_n):
    n, d_in = x.shape
    d_out = W.shape[1]
    grid = n // block_n
    return pl.pallas_call(
        _prep_body,
        grid=(grid,),
        in_specs=[
            pl.BlockSpec((block_n, d_in), lambda i: (i, 0)),
            pl.BlockSpec((d_in, d_out), lambda i: (0, 0)),
            pl.BlockSpec((2, d_out), lambda i: (0, 0)),
        ],
        out_specs=[
            pl.BlockSpec((block_n, d_out), lambda i: (i, 0)),
            pl.BlockSpec((1, 1, block_n), lambda i: (i, 0, 0)),
            pl.BlockSpec((1, 1, block_n), lambda i: (i, 0, 0)),
        ],
        out_shape=[
            jax.ShapeDtypeStruct((n, d_out), jnp.float32),
            jax.ShapeDtypeStruct((grid, 1, block_n), jnp.float32),
            jax.ShapeDtypeStruct((grid, 1, block_n), jnp.float32),
        ],
    )(x, W, a_pair)


def _sc_gat(h, s, t, ei4, ej4, n, n_pad, d, chunks_per_tile, k):
    rows_per_tile = n_pad // NS
    zc = rows_per_tile // k
    assert rows_per_tile % k == 0

    mesh = plsc.VectorSubcoreMesh(core_axis_name="c", subcore_axis_name="s")

    @functools.partial(
        pl.kernel,
        mesh=mesh,
        out_type=[
            jax.ShapeDtypeStruct((NC, n_pad, d), jnp.float32),
            jax.ShapeDtypeStruct((NC * NS, 1, LANES), jnp.float32),
        ],
        scratch_types=[
            pltpu.VMEM((n,), jnp.float32),
            pltpu.VMEM((n,), jnp.float32),
            pltpu.VMEM((2, k), jnp.int32),
            pltpu.VMEM((2, k), jnp.int32),
            pltpu.VMEM((2, k), jnp.int32),
            pltpu.VMEM((2, k), jnp.float32),
            pltpu.VMEM((2, k, d), jnp.float32),
            pltpu.VMEM((1, LANES), jnp.float32),
            pltpu.VMEM_SHARED((n_pad, d), jnp.float32),
            pltpu.SemaphoreType.DMA,
            pltpu.SemaphoreType.DMA,
            pltpu.SemaphoreType.DMA,
            pltpu.SemaphoreType.DMA,
            pltpu.SemaphoreType.DMA,
            pltpu.SemaphoreType.DMA,
            pltpu.SemaphoreType.DMA,
            pltpu.SemaphoreType.DMA,
        ],
        compiler_params=pltpu.CompilerParams(needs_layout_passes=False),
    )
    def sc_kernel(h_hbm, s_hbm, t_hbm, ei_hbm, ej_hbm, out_hbm, z_hbm,
                  s_v, t_v, ei_c, ej_c, sci, w_c, rows_v, zbuf_v, acc,
                  si0, si1, sj0, sj1, sr0, sr1, ss0, ss1):
        si = (si0, si1)
        sj = (sj0, sj1)
        sr = (sr0, sr1)
        ss = (ss0, ss1)
        cid = lax.axis_index("c")
        sid = lax.axis_index("s")
        wid = cid * NS + sid
        base = wid * chunks_per_tile

        for b in range(2):
            pltpu.async_copy(ei_hbm.at[base + b], ei_c.at[pl.ds(b, 1)], si[b])
            pltpu.async_copy(ej_hbm.at[base + b], ej_c.at[pl.ds(b, 1)], sj[b])
        pltpu.sync_copy(s_hbm, s_v)
        pltpu.sync_copy(t_hbm, t_v)

        def zbody(r, _):
            for u in range(d // LANES):
                rows_v[0, r, pl.ds(u * LANES, LANES)] = jnp.zeros(
                    (LANES,), jnp.float32)
            return 0
        lax.fori_loop(0, k, zbody, 0)
        row0 = sid * rows_per_tile
        for kk in range(zc):
            pltpu.sync_copy(rows_v.at[0], acc.at[pl.ds(row0 + kk * k, k)])

        plsc.subcore_barrier()

        def wcompute(j, b):
            for u in range(k // LANES):
                sl = pl.ds(u * LANES, LANES)
                ii = ei_c[b, sl]
                jj = ej_c[b, sl]
                l = plsc.load_gather(s_v, [ii]) + plsc.load_gather(t_v, [jj])
                l = jnp.where(l > 0, l, l * jnp.float32(0.2))
                w = jnp.exp(l)
                w_c[b, sl] = w
                zbuf_v[0, pl.ds(0, LANES)] = zbuf_v[0, pl.ds(0, LANES)] + w
                sci[b, sl] = ii

        zbuf_v[0, pl.ds(0, LANES)] = jnp.zeros((LANES,), jnp.float32)
        pltpu.make_async_copy(
            ei_hbm.at[base], ei_c.at[pl.ds(0, 1)], si[0]).wait()
        pltpu.make_async_copy(
            ej_hbm.at[base], ej_c.at[pl.ds(0, 1)], sj[0]).wait()
        pltpu.async_copy(h_hbm.at[ej_c.at[0]], rows_v.at[0], sr[0])
        wcompute(0, 0)

        def chunk_step(j, b, _unused):
            b2 = 1 - b

            pltpu.make_async_copy(
                h_hbm.at[ej_c.at[b]], rows_v.at[b], sr[b]).wait()

            @pl.when(j + 2 < chunks_per_tile)
            def _():
                pltpu.async_copy(
                    ei_hbm.at[base + j + 2], ei_c.at[pl.ds(b, 1)], si[b])
                pltpu.async_copy(
                    ej_hbm.at[base + j + 2], ej_c.at[pl.ds(b, 1)], sj[b])

            @pl.when(j + 1 < chunks_per_tile)
            def _():
                pltpu.make_async_copy(
                    ei_hbm.at[base], ei_c.at[pl.ds(b2, 1)], si[b2]).wait()
                pltpu.make_async_copy(
                    ej_hbm.at[base], ej_c.at[pl.ds(b2, 1)], sj[b2]).wait()

                @pl.when(j >= 1)
                def _():
                    pltpu.make_async_copy(
                        rows_v.at[b2], acc.at[sci.at[b2]], ss[b2]).wait()
                pltpu.async_copy(h_hbm.at[ej_c.at[b2]], rows_v.at[b2], sr[b2])
                wcompute(j + 1, b2)

            def rbody(gg, _):
                wvec = w_c[b, pl.ds(gg * LANES, LANES)]
                for rl in range(LANES):
                    ws = wvec[rl]
                    r = gg * LANES + rl
                    for u in range(d // LANES):
                        sl2 = pl.ds(u * LANES, LANES)
                        rows_v[b, r, sl2] = rows_v[b, r, sl2] * ws
                return 0
            lax.fori_loop(0, k // LANES, rbody, 0)

            pltpu.async_copy(rows_v.at[b], acc.at[sci.at[b]], ss[b], add=True)
            return 0

        npairs = chunks_per_tile // 2
        def pair_body(g, c):
            c = chunk_step(g * 2, 0, c)
            c = chunk_step(g * 2 + 1, 1, c)
            return c
        c0 = lax.fori_loop(0, npairs, pair_body, 0)
        if chunks_per_tile % 2:
            chunk_step(chunks_per_tile - 1, 0, c0)
        for b in range(2):
            pltpu.make_async_copy(
                rows_v.at[b], acc.at[sci.at[b]], ss[b]).wait()

        pltpu.sync_copy(zbuf_v, z_hbm.at[wid])

        plsc.subcore_barrier()
        pltpu.sync_copy(acc.at[pl.ds(row0, rows_per_tile)],
                        out_hbm.at[cid, pl.ds(row0, rows_per_tile)])

    return sc_kernel(h, s, t, ei4, ej4)


def _combine_body(acc_ref, z_ref, o_ref):
    p = acc_ref[0] + acc_ref[1]
    inv = jnp.float32(1.0) / jnp.sum(z_ref[...])
    v = p * inv
    o_ref[...] = jnp.concatenate([v] * HEADS, axis=1)


def _tc_combine(acc, z, n, d, block_n):
    grid = n // block_n
    return pl.pallas_call(
        _combine_body,
        grid=(grid,),
        in_specs=[
            pl.BlockSpec((NC, block_n, d), lambda i: (0, i, 0)),
            pl.BlockSpec((NC * NS, 1, LANES), lambda i: (0, 0, 0)),
        ],
        out_specs=pl.BlockSpec((block_n, HEADS * d), lambda i: (i, 0)),
        out_shape=jax.ShapeDtypeStruct((n, HEADS * d), jnp.float32),
    )(acc, z)


def kernel(x, edge_index, W, attn_weights):
    n, d_in = x.shape
    d = W.shape[1]
    e = edge_index.shape[1]

    k = 80
    chunks_per_tile = e // (NC * NS * k)
    assert chunks_per_tile * NC * NS * k == e

    a_pair = attn_weights.reshape(2, d)
    ei4 = edge_index[0].reshape(e // k, 1, k)
    ej4 = edge_index[1].reshape(e // k, 1, k)

    rpt = -(-n // NS)
    rpt = -(-rpt // 128) * 128
    n_pad = rpt * NS

    block_n = 1000
    h, s3, t3 = _tc_prep(x, W, a_pair, block_n)
    s = s3.reshape(n)
    t = t3.reshape(n)

    acc, z = _sc_gat(h, s, t, ei4, ej4, n, n_pad, d, chunks_per_tile, k)
    return _tc_combine(acc, z, n, d, block_n)


# speedup vs baseline: 13.9628x; 1.0150x over previous
import functools

import jax
import jax.numpy as jnp
from jax import lax
from jax.experimental import pallas as pl
from jax.experimental.pallas import tpu as pltpu
from jax.experimental.pallas import tpu_sc as plsc

NC = 2
NS = 16
LANES = 16
HEADS = 4


def _prep_body(x_ref, w_ref, a_ref, h_ref, s_ref, t_ref):
    h = jnp.dot(x_ref[...], w_ref[...], preferred_element_type=jnp.float32)
    h_ref[...] = h
    a = a_ref[...]
    s_ref[...] = jnp.sum(h * a[0][None, :], axis=1)[None, None, :]
    t_ref[...] = jnp.sum(h * a[1][None, :], axis=1)[None, None, :]


def _tc_prep(x, W, a_pair, block_n):
    n, d_in = x.shape
    d_out = W.shape[1]
    grid = n // block_n
    return pl.pallas_call(
        _prep_body,
        grid=(grid,),
        in_specs=[
            pl.BlockSpec((block_n, d_in), lambda i: (i, 0)),
            pl.BlockSpec((d_in, d_out), lambda i: (0, 0)),
            pl.BlockSpec((2, d_out), lambda i: (0, 0)),
        ],
        out_specs=[
            pl.BlockSpec((block_n, d_out), lambda i: (i, 0)),
            pl.BlockSpec((1, 1, block_n), lambda i: (i, 0, 0)),
            pl.BlockSpec((1, 1, block_n), lambda i: (i, 0, 0)),
        ],
        out_shape=[
            jax.ShapeDtypeStruct((n, d_out), jnp.float32),
            jax.ShapeDtypeStruct((grid, 1, block_n), jnp.float32),
            jax.ShapeDtypeStruct((grid, 1, block_n), jnp.float32),
        ],
    )(x, W, a_pair)


def _sc_gat(h, s, t, ei4, ej4, n, n_pad, d, chunks_per_tile, k):
    rows_per_tile = n_pad // NS
    zc = rows_per_tile // k
    assert rows_per_tile % k == 0

    mesh = plsc.VectorSubcoreMesh(core_axis_name="c", subcore_axis_name="s")

    @functools.partial(
        pl.kernel,
        mesh=mesh,
        out_type=[
            jax.ShapeDtypeStruct((NC, n_pad, d), jnp.float32),
            jax.ShapeDtypeStruct((NC * NS, 1, LANES), jnp.float32),
        ],
        scratch_types=[
            pltpu.VMEM((n,), jnp.float32),
            pltpu.VMEM((n,), jnp.float32),
            pltpu.VMEM((2, k), jnp.int32),
            pltpu.VMEM((2, k), jnp.int32),
            pltpu.VMEM((2, k), jnp.int32),
            pltpu.VMEM((2, k), jnp.float32),
            pltpu.VMEM((2, k, d), jnp.float32),
            pltpu.VMEM((1, LANES), jnp.float32),
            pltpu.VMEM_SHARED((n_pad, d), jnp.float32),
            pltpu.SemaphoreType.DMA,
            pltpu.SemaphoreType.DMA,
            pltpu.SemaphoreType.DMA,
            pltpu.SemaphoreType.DMA,
            pltpu.SemaphoreType.DMA,
            pltpu.SemaphoreType.DMA,
            pltpu.SemaphoreType.DMA,
            pltpu.SemaphoreType.DMA,
        ],
        compiler_params=pltpu.CompilerParams(needs_layout_passes=False),
    )
    def sc_kernel(h_hbm, s_hbm, t_hbm, ei_hbm, ej_hbm, out_hbm, z_hbm,
                  s_v, t_v, ei_c, ej_c, sci, w_c, rows_v, zbuf_v, acc,
                  si0, si1, sj0, sj1, sr0, sr1, ss0, ss1):
        si = (si0, si1)
        sj = (sj0, sj1)
        sr = (sr0, sr1)
        ss = (ss0, ss1)
        cid = lax.axis_index("c")
        sid = lax.axis_index("s")
        wid = cid * NS + sid
        base = wid * chunks_per_tile

        for b in range(2):
            pltpu.async_copy(ei_hbm.at[base + b], ei_c.at[pl.ds(b, 1)], si[b])
            pltpu.async_copy(ej_hbm.at[base + b], ej_c.at[pl.ds(b, 1)], sj[b])
        pltpu.sync_copy(s_hbm, s_v)
        pltpu.sync_copy(t_hbm, t_v)

        def zbody(r, _):
            for u in range(d // LANES):
                rows_v[0, r, pl.ds(u * LANES, LANES)] = jnp.zeros(
                    (LANES,), jnp.float32)
            return 0
        lax.fori_loop(0, k, zbody, 0)
        row0 = sid * rows_per_tile
        for kk in range(zc):
            pltpu.sync_copy(rows_v.at[0], acc.at[pl.ds(row0 + kk * k, k)])

        plsc.subcore_barrier()

        def wcompute(j, b):
            for u in range(k // LANES):
                sl = pl.ds(u * LANES, LANES)
                ii = ei_c[b, sl]
                jj = ej_c[b, sl]
                l = plsc.load_gather(s_v, [ii]) + plsc.load_gather(t_v, [jj])
                l = jnp.where(l > 0, l, l * jnp.float32(0.2))
                w = jnp.exp(l)
                w_c[b, sl] = w
                zbuf_v[0, pl.ds(0, LANES)] = zbuf_v[0, pl.ds(0, LANES)] + w
                sci[b, sl] = ii

        zbuf_v[0, pl.ds(0, LANES)] = jnp.zeros((LANES,), jnp.float32)
        pltpu.make_async_copy(
            ei_hbm.at[base], ei_c.at[pl.ds(0, 1)], si[0]).wait()
        pltpu.make_async_copy(
            ej_hbm.at[base], ej_c.at[pl.ds(0, 1)], sj[0]).wait()
        pltpu.async_copy(h_hbm.at[ej_c.at[0]], rows_v.at[0], sr[0])
        wcompute(0, 0)

        def chunk_step(j, b, _unused):
            b2 = 1 - b

            pltpu.make_async_copy(
                h_hbm.at[ej_c.at[b]], rows_v.at[b], sr[b]).wait()

            @pl.when(j + 2 < chunks_per_tile)
            def _():
                pltpu.async_copy(
                    ei_hbm.at[base + j + 2], ei_c.at[pl.ds(b, 1)], si[b])
                pltpu.async_copy(
                    ej_hbm.at[base + j + 2], ej_c.at[pl.ds(b, 1)], sj[b])

            @pl.when(j + 1 < chunks_per_tile)
            def _():
                pltpu.make_async_copy(
                    ei_hbm.at[base], ei_c.at[pl.ds(b2, 1)], si[b2]).wait()
                pltpu.make_async_copy(
                    ej_hbm.at[base], ej_c.at[pl.ds(b2, 1)], sj[b2]).wait()

                @pl.when(j >= 1)
                def _():
                    pltpu.make_async_copy(
                        rows_v.at[b2], acc.at[sci.at[b2]], ss[b2]).wait()
                pltpu.async_copy(h_hbm.at[ej_c.at[b2]], rows_v.at[b2], sr[b2])
                wcompute(j + 1, b2)

            def rbody(gg, _):
                wvec = w_c[b, pl.ds(gg * LANES, LANES)]
                for rl in range(LANES):
                    ws = wvec[rl]
                    r = gg * LANES + rl
                    for u in range(d // LANES):
                        sl2 = pl.ds(u * LANES, LANES)
                        rows_v[b, r, sl2] = rows_v[b, r, sl2] * ws
                return 0
            lax.fori_loop(0, k // LANES, rbody, 0)

            pltpu.async_copy(rows_v.at[b], acc.at[sci.at[b]], ss[b], add=True)
            return 0

        npairs = chunks_per_tile // 2
        def pair_body(g, c):
            c = chunk_step(g * 2, 0, c)
            c = chunk_step(g * 2 + 1, 1, c)
            return c
        c0 = lax.fori_loop(0, npairs, pair_body, 0)
        if chunks_per_tile % 2:
            chunk_step(chunks_per_tile - 1, 0, c0)
        for b in range(2):
            pltpu.make_async_copy(
                rows_v.at[b], acc.at[sci.at[b]], ss[b]).wait()

        pltpu.sync_copy(zbuf_v, z_hbm.at[wid])

        plsc.subcore_barrier()
        pltpu.sync_copy(acc.at[pl.ds(row0, rows_per_tile)],
                        out_hbm.at[cid, pl.ds(row0, rows_per_tile)])

    return sc_kernel(h, s, t, ei4, ej4)


def _combine_body(acc_ref, z_ref, o_ref):
    p = acc_ref[0] + acc_ref[1]
    inv = jnp.float32(1.0) / jnp.sum(z_ref[...])
    v = p * inv
    o_ref[...] = jnp.concatenate([v] * HEADS, axis=1)


def _tc_combine(acc, z, n, d, block_n):
    grid = n // block_n
    return pl.pallas_call(
        _combine_body,
        grid=(grid,),
        in_specs=[
            pl.BlockSpec((NC, block_n, d), lambda i: (0, i, 0)),
            pl.BlockSpec((NC * NS, 1, LANES), lambda i: (0, 0, 0)),
        ],
        out_specs=pl.BlockSpec((block_n, HEADS * d), lambda i: (i, 0)),
        out_shape=jax.ShapeDtypeStruct((n, HEADS * d), jnp.float32),
    )(acc, z)


def kernel(x, edge_index, W, attn_weights):
    n, d_in = x.shape
    d = W.shape[1]
    e = edge_index.shape[1]

    k = 80
    chunks_per_tile = e // (NC * NS * k)
    assert chunks_per_tile * NC * NS * k == e

    a_pair = attn_weights.reshape(2, d)
    ei4 = edge_index[0].reshape(e // k, 1, k)
    ej4 = edge_index[1].reshape(e // k, 1, k)

    rpt = -(-n // NS)
    rpt = -(-rpt // 128) * 128
    n_pad = rpt * NS

    block_n = 2000
    h, s3, t3 = _tc_prep(x, W, a_pair, block_n)
    s = s3.reshape(n)
    t = t3.reshape(n)

    acc, z = _sc_gat(h, s, t, ei4, ej4, n, n_pad, d, chunks_per_tile, k)
    return _tc_combine(acc, z, n, d, block_n)
